# baseline (speedup 1.0000x reference)
import functools

import jax
import jax.numpy as jnp
from jax import lax
from jax.experimental import pallas as pl
from jax.experimental.pallas import tpu as pltpu
from jax.experimental.pallas import tpu_sc as plsc

N, E, D, H = 10000, 320000, 128, 256
NP = 10240
NH = NP // 128
NC, NS = 2, 16
L = 16
CH = 128
E2 = 327680
NB = E2 // NC // NS // CH
CH2 = 96
NB2 = 105
E2B = NC * NS * NB2 * CH2

_mesh = plsc.VectorSubcoreMesh(core_axis_name="c", subcore_axis_name="s")


def _count_chunk(idx3, t, hist_v):
    for k in range(CH // L):
        d16 = idx3[t, 1, pl.ds(k * L, L)]
        cnt, last = plsc.scan_count(d16)
        plsc.addupdate_scatter(
            hist_v,
            [lax.shift_right_logical(d16, 7), lax.bitwise_and(d16, 127)],
            cnt.astype(jnp.float32),
            mask=last,
        )


def _edge_pipe(tab, ec_hbm, cbase, acc_sh, idxb, rowsb, semi, semg, sems,
               n, nd, ni, hist_v=None):
    def istart(j):
        pltpu.async_copy(ec_hbm.at[cbase + j], idxb.at[j % ni],
                         semi.at[j % ni])

    def iwait(j):
        pltpu.make_async_copy(ec_hbm.at[cbase + j], idxb.at[j % ni],
                              semi.at[j % ni]).wait()

    def gstart(j):
        pltpu.async_copy(tab.at[idxb.at[j % ni, 0]], rowsb.at[j % nd],
                         semg.at[j % nd])

    def gwait(j):
        pltpu.make_async_copy(tab.at[idxb.at[j % ni, 0]], rowsb.at[j % nd],
                              semg.at[j % nd]).wait()

    def sstart(j):
        pltpu.async_copy(rowsb.at[j % nd], acc_sh.at[idxb.at[j % ni, 1]],
                         sems.at[j % nd], add=True)

    def swait(j):
        pltpu.make_async_copy(rowsb.at[j % nd], acc_sh.at[idxb.at[j % ni, 1]],
                              sems.at[j % nd]).wait()

    istart(0)
    istart(1)
    iwait(0)
    gstart(0)

    def body(j, carry):
        gwait(j)

        @pl.when(j >= nd - 1)
        def _():
            swait(j - (nd - 1))

        @pl.when(j + 1 < n)
        def _():
            iwait(j + 1)
            gstart(j + 1)

        sstart(j)

        @pl.when(j + 2 < n)
        def _():
            istart(j + 2)

        if hist_v is not None:
            _count_chunk(idxb, j % ni, hist_v)
        return carry

    lax.fori_loop(0, n, body, 0)
    for k in range(nd - 1):
        swait(n - 1 - k)


def _agg1_body(xp_hbm, ec_hbm, zeros_hbm, out_hbm, cnt_hbm,
               idx3, rows2, iota_v, hist_v, semi, semg, sems,
               acc_sh, cnt_sh):
    c = lax.axis_index("c")
    s = lax.axis_index("s")
    rows = NP // NS
    rbase = s * rows
    cbase = (c * NS + s) * NB
    pltpu.sync_copy(zeros_hbm.at[pl.ds(rbase, rows)],
                    acc_sh.at[pl.ds(rbase, rows)])
    pltpu.sync_copy(zeros_hbm.at[pl.ds(0, NH)], hist_v)

    @pl.when(s == 0)
    def _():
        pltpu.sync_copy(zeros_hbm.at[pl.ds(0, NH)], cnt_sh)

    for k in range(NH // L):
        iota_v[0, pl.ds(k * L, L)] = lax.iota(jnp.int32, L) + k * L
    plsc.subcore_barrier()

    _edge_pipe(xp_hbm, ec_hbm, cbase, acc_sh, idx3, rows2, semi, semg, sems,
               NB, 2, 3, hist_v=hist_v)
    pltpu.sync_copy(hist_v, cnt_sh.at[iota_v.at[0]], add=True)
    plsc.subcore_barrier()
    pltpu.sync_copy(acc_sh.at[pl.ds(rbase, rows)],
                    out_hbm.at[c].at[pl.ds(rbase, rows)])

    @pl.when(s == 0)
    def _():
        pltpu.sync_copy(cnt_sh, cnt_hbm.at[c])


_agg1 = functools.partial(
    pl.kernel,
    out_type=(jax.ShapeDtypeStruct((NC, NP, D), jnp.float32),
              jax.ShapeDtypeStruct((NC, NH, 128), jnp.float32)),
    mesh=_mesh,
    scratch_types=[
        pltpu.VMEM((3, 2, CH), jnp.int32),
        pltpu.VMEM((2, CH, D), jnp.float32),
        pltpu.VMEM((1, NH), jnp.int32),
        pltpu.VMEM((NH, 128), jnp.float32),
        pltpu.SemaphoreType.DMA((3,)),
        pltpu.SemaphoreType.DMA((2,)),
        pltpu.SemaphoreType.DMA((2,)),
        pltpu.VMEM_SHARED((NP, D), jnp.float32),
        pltpu.VMEM_SHARED((NH, 128), jnp.float32),
    ],
    compiler_params=pltpu.CompilerParams(needs_layout_passes=False),
)(_agg1_body)


def _agg2_body(h1a_hbm, h1b_hbm, ec_hbm, zeros_hbm, out_hbm,
               idx3, rows2, semi, semg, sems, acc_sh):
    c = lax.axis_index("c")
    s = lax.axis_index("s")
    rows = NP // NS
    rbase = s * rows
    cbase = (c * NS + s) * NB2
    for half, tab in ((0, h1a_hbm), (1, h1b_hbm)):
        pltpu.sync_copy(zeros_hbm.at[pl.ds(rbase, rows)],
                        acc_sh.at[pl.ds(rbase, rows)])
        plsc.subcore_barrier()
        _edge_pipe(tab, ec_hbm, cbase, acc_sh, idx3, rows2, semi, semg, sems,
                   NB2, 3, 4)
        plsc.subcore_barrier()
        pltpu.sync_copy(acc_sh.at[pl.ds(rbase, rows)],
                        out_hbm.at[c].at[half].at[pl.ds(rbase, rows)])


_agg2 = functools.partial(
    pl.kernel,
    out_type=jax.ShapeDtypeStruct((NC, 2, NP, D), jnp.float32),
    mesh=_mesh,
    scratch_types=[
        pltpu.VMEM((4, 2, CH2), jnp.int32),
        pltpu.VMEM((3, CH2, D), jnp.float32),
        pltpu.SemaphoreType.DMA((4,)),
        pltpu.SemaphoreType.DMA((3,)),
        pltpu.SemaphoreType.DMA((3,)),
        pltpu.VMEM_SHARED((NP, D), jnp.float32),
    ],
    compiler_params=pltpu.CompilerParams(needs_layout_passes=False),
)(_agg2_body)


BN = 1024


def _tc1_body(acc_ref, cnt_ref, x_ref, w1l_ref, w1r_ref, c1_ref, w2r_ref,
              h1a_ref, h1b_ref, z_ref, inv_ref):
    agg = acc_ref[0] + acc_ref[1]
    hist = cnt_ref[0] + cnt_ref[1]
    rsel = lax.broadcasted_iota(jnp.int32, (BN, BN // 128), 0) >> 7
    csel = lax.broadcasted_iota(jnp.int32, (BN, BN // 128), 1)
    onehot = jnp.where(rsel == csel, 1.0, 0.0)
    rep = jnp.dot(onehot, hist, preferred_element_type=jnp.float32)
    ridx = lax.broadcasted_iota(jnp.int32, (BN, 128), 0) & 127
    lidx = lax.broadcasted_iota(jnp.int32, (BN, 128), 1)
    cnt = jnp.sum(jnp.where(ridx == lidx, rep, 0.0), axis=1, keepdims=True)
    inv = 1.0 / jnp.maximum(cnt, 1.0)
    mean = agg * inv
    lin = (jnp.dot(mean, w1l_ref[...], preferred_element_type=jnp.float32)
           + jnp.dot(x_ref[...], w1r_ref[...], preferred_element_type=jnp.float32)
           + c1_ref[...])
    h1 = jnp.maximum(lin, 0.0)
    h1a_ref[...] = h1[:, :D]
    h1b_ref[...] = h1[:, D:]
    z_ref[...] = jnp.dot(h1, w2r_ref[...], preferred_element_type=jnp.float32)
    inv_ref[...] = inv


_tc1 = pl.pallas_call(
    _tc1_body,
    grid=(NP // BN,),
    in_specs=[
        pl.BlockSpec((NC, BN, D), lambda i: (0, i, 0)),
        pl.BlockSpec((NC, BN // 128, 128), lambda i: (0, i, 0)),
        pl.BlockSpec((BN, D), lambda i: (i, 0)),
        pl.BlockSpec((D, H), lambda i: (0, 0)),
        pl.BlockSpec((D, H), lambda i: (0, 0)),
        pl.BlockSpec((1, H), lambda i: (0, 0)),
        pl.BlockSpec((H, H), lambda i: (0, 0)),
    ],
    out_specs=[
        pl.BlockSpec((BN, D), lambda i: (i, 0)),
        pl.BlockSpec((BN, D), lambda i: (i, 0)),
        pl.BlockSpec((BN, H), lambda i: (i, 0)),
        pl.BlockSpec((BN, 1), lambda i: (i, 0)),
    ],
    out_shape=[
        jax.ShapeDtypeStruct((NP, D), jnp.float32),
        jax.ShapeDtypeStruct((NP, D), jnp.float32),
        jax.ShapeDtypeStruct((NP, H), jnp.float32),
        jax.ShapeDtypeStruct((NP, 1), jnp.float32),
    ],
)


BN2 = 1000


def _tc2_body(acc2_ref, inv_ref, z_ref, w2l_ref, c2_ref, out_ref):
    inv = inv_ref[...]
    m0 = (acc2_ref[0, 0] + acc2_ref[1, 0]) * inv
    m1 = (acc2_ref[0, 1] + acc2_ref[1, 1]) * inv
    lin = (jnp.dot(m0, w2l_ref[0], preferred_element_type=jnp.float32)
           + jnp.dot(m1, w2l_ref[1], preferred_element_type=jnp.float32)
           + z_ref[...] + c2_ref[...])
    out_ref[...] = jnp.maximum(lin, 0.0)


_tc2 = pl.pallas_call(
    _tc2_body,
    grid=(N // BN2,),
    in_specs=[
        pl.BlockSpec((NC, 2, BN2, D), lambda i: (0, 0, i, 0)),
        pl.BlockSpec((BN2, 1), lambda i: (i, 0)),
        pl.BlockSpec((BN2, H), lambda i: (i, 0)),
        pl.BlockSpec((2, D, H), lambda i: (0, 0, 0)),
        pl.BlockSpec((1, H), lambda i: (0, 0)),
    ],
    out_specs=pl.BlockSpec((BN2, H), lambda i: (i, 0)),
    out_shape=jax.ShapeDtypeStruct((N, H), jnp.float32),
)


def kernel(x, edge_index, W1l, b1, W1r, bn1_g, bn1_b, W2l, b2, W2r,
           bn2_g, bn2_b):
    ei = edge_index.astype(jnp.int32)

    def chunked(n_pad, ch):
        pad = n_pad - E
        src_p = jnp.concatenate(
            [ei[0], jnp.arange(pad, dtype=jnp.int32) % N])
        dst_p = jnp.concatenate(
            [ei[1], N + (jnp.arange(pad, dtype=jnp.int32) % (NP - N))])
        return jnp.stack(
            [src_p.reshape(n_pad // ch, ch), dst_p.reshape(n_pad // ch, ch)],
            axis=1)

    ec = chunked(E2, CH)
    ecb = chunked(E2B, CH2)
    bn_inv = 1.0 / jnp.sqrt(1.0 + 1e-5)
    s1 = bn1_g * bn_inv
    s2 = bn2_g * bn_inv
    c1 = (b1 * s1 + bn1_b).reshape(1, H)
    c2 = (b2 * s2 + bn2_b).reshape(1, H)
    w1l = W1l.T * s1[None, :]
    w1r = W1r.T * s1[None, :]
    w2r = W2r.T * s2[None, :]
    w2l = (W2l.T * s2[None, :]).reshape(2, D, H)

    xp = jnp.concatenate([x, jnp.zeros((NP - N, D), jnp.float32)], axis=0)
    zeros_h = jnp.zeros((NP, D), jnp.float32)

    acc1, cnt = _agg1(xp, ec, zeros_h)
    h1a, h1b, z, inv = _tc1(acc1, cnt, xp, w1l, w1r, c1, w2r)
    acc2 = _agg2(h1a, h1b, ecb, zeros_h)
    return _tc2(acc2, inv, z, w2l, c2)


# speedup vs baseline: 1.0643x; 1.0643x over previous
import functools

import jax
import jax.numpy as jnp
from jax import lax
from jax.experimental import pallas as pl
from jax.experimental.pallas import tpu as pltpu
from jax.experimental.pallas import tpu_sc as plsc

N, E, D, H = 10000, 320000, 128, 256
NP = 10240
NH = NP // 128
NC, NS = 2, 16
L = 16
CH = 128
E2 = 327680
NB = E2 // NC // NS // CH


_mesh = plsc.VectorSubcoreMesh(core_axis_name="c", subcore_axis_name="s")


def _count_chunk(idx3, t, hist_v):
    for k in range(CH // L):
        d16 = idx3[t, 1, pl.ds(k * L, L)]
        cnt, last = plsc.scan_count(d16)
        plsc.addupdate_scatter(
            hist_v,
            [lax.shift_right_logical(d16, 7), lax.bitwise_and(d16, 127)],
            cnt.astype(jnp.float32),
            mask=last,
        )


def _edge_pipe(tab, ec_hbm, cbase, acc_sh, idxb, rowsb, semi, semg, sems,
               n, nd, ni, hist_v=None):
    def istart(j):
        pltpu.async_copy(ec_hbm.at[cbase + j], idxb.at[j % ni],
                         semi.at[j % ni])

    def iwait(j):
        pltpu.make_async_copy(ec_hbm.at[cbase + j], idxb.at[j % ni],
                              semi.at[j % ni]).wait()

    def gstart(j):
        pltpu.async_copy(tab.at[idxb.at[j % ni, 0]], rowsb.at[j % nd],
                         semg.at[j % nd])

    def gwait(j):
        pltpu.make_async_copy(tab.at[idxb.at[j % ni, 0]], rowsb.at[j % nd],
                              semg.at[j % nd]).wait()

    def sstart(j):
        pltpu.async_copy(rowsb.at[j % nd], acc_sh.at[idxb.at[j % ni, 1]],
                         sems.at[j % nd], add=True)

    def swait(j):
        pltpu.make_async_copy(rowsb.at[j % nd], acc_sh.at[idxb.at[j % ni, 1]],
                              sems.at[j % nd]).wait()

    istart(0)
    istart(1)
    iwait(0)
    gstart(0)

    def body(j, carry):
        gwait(j)

        @pl.when(j >= nd - 1)
        def _():
            swait(j - (nd - 1))

        @pl.when(j + 1 < n)
        def _():
            iwait(j + 1)
            gstart(j + 1)

        sstart(j)

        @pl.when(j + 2 < n)
        def _():
            istart(j + 2)

        if hist_v is not None:
            _count_chunk(idxb, j % ni, hist_v)
        return carry

    lax.fori_loop(0, n, body, 0)
    for k in range(nd - 1):
        swait(n - 1 - k)


def _agg1_body(xp_hbm, ec_hbm, zeros_hbm, out_hbm, cnt_hbm,
               idx3, rows2, iota_v, hist_v, semi, semg, sems,
               acc_sh, cnt_sh):
    c = lax.axis_index("c")
    s = lax.axis_index("s")
    rows = NP // NS
    rbase = s * rows
    cbase = (c * NS + s) * NB
    pltpu.sync_copy(zeros_hbm.at[pl.ds(rbase, rows)],
                    acc_sh.at[pl.ds(rbase, rows)])
    pltpu.sync_copy(zeros_hbm.at[pl.ds(0, NH)], hist_v)

    @pl.when(s == 0)
    def _():
        pltpu.sync_copy(zeros_hbm.at[pl.ds(0, NH)], cnt_sh)

    for k in range(NH // L):
        iota_v[0, pl.ds(k * L, L)] = lax.iota(jnp.int32, L) + k * L
    plsc.subcore_barrier()

    _edge_pipe(xp_hbm, ec_hbm, cbase, acc_sh, idx3, rows2, semi, semg, sems,
               NB, 2, 3, hist_v=hist_v)
    pltpu.sync_copy(hist_v, cnt_sh.at[iota_v.at[0]], add=True)
    plsc.subcore_barrier()
    pltpu.sync_copy(acc_sh.at[pl.ds(rbase, rows)],
                    out_hbm.at[c].at[pl.ds(rbase, rows)])

    @pl.when(s == 0)
    def _():
        pltpu.sync_copy(cnt_sh, cnt_hbm.at[c])


_agg1 = functools.partial(
    pl.kernel,
    out_type=(jax.ShapeDtypeStruct((NC, NP, D), jnp.float32),
              jax.ShapeDtypeStruct((NC, NH, 128), jnp.float32)),
    mesh=_mesh,
    scratch_types=[
        pltpu.VMEM((3, 2, CH), jnp.int32),
        pltpu.VMEM((2, CH, D), jnp.float32),
        pltpu.VMEM((1, NH), jnp.int32),
        pltpu.VMEM((NH, 128), jnp.float32),
        pltpu.SemaphoreType.DMA((3,)),
        pltpu.SemaphoreType.DMA((2,)),
        pltpu.SemaphoreType.DMA((2,)),
        pltpu.VMEM_SHARED((NP, D), jnp.float32),
        pltpu.VMEM_SHARED((NH, 128), jnp.float32),
    ],
    compiler_params=pltpu.CompilerParams(needs_layout_passes=False),
)(_agg1_body)


def _agg2_body(h1a_hbm, h1b_hbm, ec_hbm, zeros_hbm, out_hbm,
               idx3, rows2, semi, semg, sems, acc_sh):
    c = lax.axis_index("c")
    s = lax.axis_index("s")
    rows = NP // NS
    rbase = s * rows
    cbase = (c * NS + s) * NB
    for half, tab in ((0, h1a_hbm), (1, h1b_hbm)):
        pltpu.sync_copy(zeros_hbm.at[pl.ds(rbase, rows)],
                        acc_sh.at[pl.ds(rbase, rows)])
        plsc.subcore_barrier()
        _edge_pipe(tab, ec_hbm, cbase, acc_sh, idx3, rows2, semi, semg, sems,
                   NB, 2, 3)
        plsc.subcore_barrier()
        pltpu.sync_copy(acc_sh.at[pl.ds(rbase, rows)],
                        out_hbm.at[c].at[half].at[pl.ds(rbase, rows)])


_agg2 = functools.partial(
    pl.kernel,
    out_type=jax.ShapeDtypeStruct((NC, 2, NP, D), jnp.float32),
    mesh=_mesh,
    scratch_types=[
        pltpu.VMEM((3, 2, CH), jnp.int32),
        pltpu.VMEM((2, CH, D), jnp.float32),
        pltpu.SemaphoreType.DMA((3,)),
        pltpu.SemaphoreType.DMA((2,)),
        pltpu.SemaphoreType.DMA((2,)),
        pltpu.VMEM_SHARED((NP, D), jnp.float32),
    ],
    compiler_params=pltpu.CompilerParams(needs_layout_passes=False),
)(_agg2_body)


BN = 1024


def _tc1_body(acc_ref, cnt_ref, x_ref, w1l_ref, w1r_ref, c1_ref, w2r_ref,
              h1a_ref, h1b_ref, z_ref, inv_ref):
    agg = acc_ref[0] + acc_ref[1]
    hist = cnt_ref[0] + cnt_ref[1]
    rsel = lax.broadcasted_iota(jnp.int32, (BN, BN // 128), 0) >> 7
    csel = lax.broadcasted_iota(jnp.int32, (BN, BN // 128), 1)
    onehot = jnp.where(rsel == csel, 1.0, 0.0)
    rep = jnp.dot(onehot, hist, preferred_element_type=jnp.float32)
    ridx = lax.broadcasted_iota(jnp.int32, (BN, 128), 0) & 127
    lidx = lax.broadcasted_iota(jnp.int32, (BN, 128), 1)
    cnt = jnp.sum(jnp.where(ridx == lidx, rep, 0.0), axis=1, keepdims=True)
    inv = 1.0 / jnp.maximum(cnt, 1.0)
    mean = agg * inv
    lin = (jnp.dot(mean, w1l_ref[...], preferred_element_type=jnp.float32)
           + jnp.dot(x_ref[...], w1r_ref[...], preferred_element_type=jnp.float32)
           + c1_ref[...])
    h1 = jnp.maximum(lin, 0.0)
    h1a_ref[...] = h1[:, :D]
    h1b_ref[...] = h1[:, D:]
    z_ref[...] = jnp.dot(h1, w2r_ref[...], preferred_element_type=jnp.float32)
    inv_ref[...] = inv


_tc1 = pl.pallas_call(
    _tc1_body,
    grid=(NP // BN,),
    in_specs=[
        pl.BlockSpec((NC, BN, D), lambda i: (0, i, 0)),
        pl.BlockSpec((NC, BN // 128, 128), lambda i: (0, i, 0)),
        pl.BlockSpec((BN, D), lambda i: (i, 0)),
        pl.BlockSpec((D, H), lambda i: (0, 0)),
        pl.BlockSpec((D, H), lambda i: (0, 0)),
        pl.BlockSpec((1, H), lambda i: (0, 0)),
        pl.BlockSpec((H, H), lambda i: (0, 0)),
    ],
    out_specs=[
        pl.BlockSpec((BN, D), lambda i: (i, 0)),
        pl.BlockSpec((BN, D), lambda i: (i, 0)),
        pl.BlockSpec((BN, H), lambda i: (i, 0)),
        pl.BlockSpec((BN, 1), lambda i: (i, 0)),
    ],
    out_shape=[
        jax.ShapeDtypeStruct((NP, D), jnp.float32),
        jax.ShapeDtypeStruct((NP, D), jnp.float32),
        jax.ShapeDtypeStruct((NP, H), jnp.float32),
        jax.ShapeDtypeStruct((NP, 1), jnp.float32),
    ],
)


BN2 = 1000


def _tc2_body(acc2_ref, inv_ref, z_ref, w2l_ref, c2_ref, out_ref):
    inv = inv_ref[...]
    m0 = (acc2_ref[0, 0] + acc2_ref[1, 0]) * inv
    m1 = (acc2_ref[0, 1] + acc2_ref[1, 1]) * inv
    lin = (jnp.dot(m0, w2l_ref[0], preferred_element_type=jnp.float32)
           + jnp.dot(m1, w2l_ref[1], preferred_element_type=jnp.float32)
           + z_ref[...] + c2_ref[...])
    out_ref[...] = jnp.maximum(lin, 0.0)


_tc2 = pl.pallas_call(
    _tc2_body,
    grid=(N // BN2,),
    in_specs=[
        pl.BlockSpec((NC, 2, BN2, D), lambda i: (0, 0, i, 0)),
        pl.BlockSpec((BN2, 1), lambda i: (i, 0)),
        pl.BlockSpec((BN2, H), lambda i: (i, 0)),
        pl.BlockSpec((2, D, H), lambda i: (0, 0, 0)),
        pl.BlockSpec((1, H), lambda i: (0, 0)),
    ],
    out_specs=pl.BlockSpec((BN2, H), lambda i: (i, 0)),
    out_shape=jax.ShapeDtypeStruct((N, H), jnp.float32),
)


def kernel(x, edge_index, W1l, b1, W1r, bn1_g, bn1_b, W2l, b2, W2r,
           bn2_g, bn2_b):
    ei = edge_index.astype(jnp.int32)

    def chunked(n_pad, ch):
        pad = n_pad - E
        src_p = jnp.concatenate(
            [ei[0], jnp.arange(pad, dtype=jnp.int32) % N])
        dst_p = jnp.concatenate(
            [ei[1], N + (jnp.arange(pad, dtype=jnp.int32) % (NP - N))])
        return jnp.stack(
            [src_p.reshape(n_pad // ch, ch), dst_p.reshape(n_pad // ch, ch)],
            axis=1)

    ec = chunked(E2, CH)
    bn_inv = 1.0 / jnp.sqrt(1.0 + 1e-5)
    s1 = bn1_g * bn_inv
    s2 = bn2_g * bn_inv
    c1 = (b1 * s1 + bn1_b).reshape(1, H)
    c2 = (b2 * s2 + bn2_b).reshape(1, H)
    w1l = W1l.T * s1[None, :]
    w1r = W1r.T * s1[None, :]
    w2r = W2r.T * s2[None, :]
    w2l = (W2l.T * s2[None, :]).reshape(2, D, H)

    xp = jnp.concatenate([x, jnp.zeros((NP - N, D), jnp.float32)], axis=0)
    zeros_h = jnp.zeros((NP, D), jnp.float32)

    acc1, cnt = _agg1(xp, ec, zeros_h)
    h1a, h1b, z, inv = _tc1(acc1, cnt, xp, w1l, w1r, c1, w2r)
    acc2 = _agg2(h1a, h1b, ec, zeros_h)
    return _tc2(acc2, inv, z, w2l, c2)


# speedup vs baseline: 1.0967x; 1.0305x over previous
import functools

import jax
import jax.numpy as jnp
from jax import lax
from jax.experimental import pallas as pl
from jax.experimental.pallas import tpu as pltpu
from jax.experimental.pallas import tpu_sc as plsc

N, E, D, H = 10000, 320000, 128, 256
NP = 10240
NH = NP // 128
NC, NS = 2, 16
L = 16
CH = 128
E2 = 327680
NB = E2 // NC // NS // CH


_mesh = plsc.VectorSubcoreMesh(core_axis_name="c", subcore_axis_name="s")


def _count_chunk(idx3, t, hist_v):
    for k in range(CH // L):
        d16 = idx3[t, 1, pl.ds(k * L, L)]
        cnt, last = plsc.scan_count(d16)
        plsc.addupdate_scatter(
            hist_v,
            [lax.shift_right_logical(d16, 7), lax.bitwise_and(d16, 127)],
            cnt.astype(jnp.float32),
            mask=last,
        )


def _edge_pipe(tab, ec_hbm, cbase, acc_sh, idxb, rowsb, semi, semg, sems,
               n, nd, ni, hist_v=None):
    def istart(j):
        pltpu.async_copy(ec_hbm.at[cbase + j], idxb.at[j % ni],
                         semi.at[j % ni])

    def iwait(j):
        pltpu.make_async_copy(ec_hbm.at[cbase + j], idxb.at[j % ni],
                              semi.at[j % ni]).wait()

    def gstart(j):
        pltpu.async_copy(tab.at[idxb.at[j % ni, 0]], rowsb.at[j % nd],
                         semg.at[j % nd])

    def gwait(j):
        pltpu.make_async_copy(tab.at[idxb.at[j % ni, 0]], rowsb.at[j % nd],
                              semg.at[j % nd]).wait()

    def sstart(j):
        pltpu.async_copy(rowsb.at[j % nd], acc_sh.at[idxb.at[j % ni, 1]],
                         sems.at[j % nd], add=True)

    def swait(j):
        pltpu.make_async_copy(rowsb.at[j % nd], acc_sh.at[idxb.at[j % ni, 1]],
                              sems.at[j % nd]).wait()

    istart(0)
    istart(1)
    iwait(0)
    gstart(0)

    def body(j, carry):
        gwait(j)

        @pl.when(j >= nd - 1)
        def _():
            swait(j - (nd - 1))

        @pl.when(j + 1 < n)
        def _():
            iwait(j + 1)
            gstart(j + 1)

        sstart(j)

        @pl.when(j + 2 < n)
        def _():
            istart(j + 2)

        if hist_v is not None:
            _count_chunk(idxb, j % ni, hist_v)
        return carry

    lax.fori_loop(0, n, body, 0)
    for k in range(nd - 1):
        swait(n - 1 - k)


def _agg1_body(xp_hbm, ec_hbm, zeros_hbm, out_hbm, cnt_hbm,
               idx3, rows2, iota_v, hist_v, semi, semg, sems,
               acc_sh, cnt_sh):
    c = lax.axis_index("c")
    s = lax.axis_index("s")
    rows = NP // NS
    rbase = s * rows
    cbase = (c * NS + s) * NB
    pltpu.sync_copy(zeros_hbm.at[pl.ds(rbase, rows)],
                    acc_sh.at[pl.ds(rbase, rows)])
    pltpu.sync_copy(zeros_hbm.at[pl.ds(0, NH)], hist_v)

    @pl.when(s == 0)
    def _():
        pltpu.sync_copy(zeros_hbm.at[pl.ds(0, NH)], cnt_sh)

    for k in range(NH // L):
        iota_v[0, pl.ds(k * L, L)] = lax.iota(jnp.int32, L) + k * L
    plsc.subcore_barrier()

    _edge_pipe(xp_hbm, ec_hbm, cbase, acc_sh, idx3, rows2, semi, semg, sems,
               NB, 2, 3, hist_v=hist_v)
    pltpu.sync_copy(hist_v, cnt_sh.at[iota_v.at[0]], add=True)
    plsc.subcore_barrier()
    pltpu.sync_copy(acc_sh.at[pl.ds(rbase, rows)],
                    out_hbm.at[c].at[pl.ds(rbase, rows)])

    @pl.when(s == 0)
    def _():
        pltpu.sync_copy(cnt_sh, cnt_hbm.at[c])


_agg1 = functools.partial(
    pl.kernel,
    out_type=(jax.ShapeDtypeStruct((NC, NP, D), jnp.float32),
              jax.ShapeDtypeStruct((NC, NH, 128), jnp.float32)),
    mesh=_mesh,
    scratch_types=[
        pltpu.VMEM((3, 2, CH), jnp.int32),
        pltpu.VMEM((2, CH, D), jnp.float32),
        pltpu.VMEM((1, NH), jnp.int32),
        pltpu.VMEM((NH, 128), jnp.float32),
        pltpu.SemaphoreType.DMA((3,)),
        pltpu.SemaphoreType.DMA((2,)),
        pltpu.SemaphoreType.DMA((2,)),
        pltpu.VMEM_SHARED((NP, D), jnp.float32),
        pltpu.VMEM_SHARED((NH, 128), jnp.float32),
    ],
    compiler_params=pltpu.CompilerParams(needs_layout_passes=False),
)(_agg1_body)


def _agg2_body(h1s_hbm, ec_hbm, zeros_hbm, out_hbm,
               idx3, rows2, semi, semg, sems, acc_sh):
    c = lax.axis_index("c")
    s = lax.axis_index("s")
    rows = NP // NS
    rbase = s * rows
    cbase = s * (2 * NB)
    pltpu.sync_copy(zeros_hbm.at[pl.ds(rbase, rows)],
                    acc_sh.at[pl.ds(rbase, rows)])
    plsc.subcore_barrier()
    _edge_pipe(h1s_hbm.at[c], ec_hbm, cbase, acc_sh, idx3, rows2,
               semi, semg, sems, 2 * NB, 2, 3)
    plsc.subcore_barrier()
    pltpu.sync_copy(acc_sh.at[pl.ds(rbase, rows)],
                    out_hbm.at[c].at[pl.ds(rbase, rows)])


_agg2 = functools.partial(
    pl.kernel,
    out_type=jax.ShapeDtypeStruct((NC, NP, D), jnp.float32),
    mesh=_mesh,
    scratch_types=[
        pltpu.VMEM((3, 2, CH), jnp.int32),
        pltpu.VMEM((2, CH, D), jnp.float32),
        pltpu.SemaphoreType.DMA((3,)),
        pltpu.SemaphoreType.DMA((2,)),
        pltpu.SemaphoreType.DMA((2,)),
        pltpu.VMEM_SHARED((NP, D), jnp.float32),
    ],
    compiler_params=pltpu.CompilerParams(needs_layout_passes=False),
)(_agg2_body)


BN = 1024


def _tc1_body(acc_ref, cnt_ref, x_ref, w1l_ref, w1r_ref, c1_ref, w2r_ref,
              h1s_ref, z_ref, inv_ref):
    agg = acc_ref[0] + acc_ref[1]
    hist = cnt_ref[0] + cnt_ref[1]
    rsel = lax.broadcasted_iota(jnp.int32, (BN, BN // 128), 0) >> 7
    csel = lax.broadcasted_iota(jnp.int32, (BN, BN // 128), 1)
    onehot = jnp.where(rsel == csel, 1.0, 0.0)
    rep = jnp.dot(onehot, hist, preferred_element_type=jnp.float32)
    ridx = lax.broadcasted_iota(jnp.int32, (BN, 128), 0) & 127
    lidx = lax.broadcasted_iota(jnp.int32, (BN, 128), 1)
    cnt = jnp.sum(jnp.where(ridx == lidx, rep, 0.0), axis=1, keepdims=True)
    inv = 1.0 / jnp.maximum(cnt, 1.0)
    mean = agg * inv
    lin = (jnp.dot(mean, w1l_ref[...], preferred_element_type=jnp.float32)
           + jnp.dot(x_ref[...], w1r_ref[...], preferred_element_type=jnp.float32)
           + c1_ref[...])
    h1 = jnp.maximum(lin, 0.0)
    h1s_ref[0] = h1[:, :D]
    h1s_ref[1] = h1[:, D:]
    z_ref[...] = jnp.dot(h1, w2r_ref[...], preferred_element_type=jnp.float32)
    inv_ref[...] = inv


_tc1 = pl.pallas_call(
    _tc1_body,
    grid=(NP // BN,),
    in_specs=[
        pl.BlockSpec((NC, BN, D), lambda i: (0, i, 0)),
        pl.BlockSpec((NC, BN // 128, 128), lambda i: (0, i, 0)),
        pl.BlockSpec((BN, D), lambda i: (i, 0)),
        pl.BlockSpec((D, H), lambda i: (0, 0)),
        pl.BlockSpec((D, H), lambda i: (0, 0)),
        pl.BlockSpec((1, H), lambda i: (0, 0)),
        pl.BlockSpec((H, H), lambda i: (0, 0)),
    ],
    out_specs=[
        pl.BlockSpec((2, BN, D), lambda i: (0, i, 0)),
        pl.BlockSpec((BN, H), lambda i: (i, 0)),
        pl.BlockSpec((BN, 1), lambda i: (i, 0)),
    ],
    out_shape=[
        jax.ShapeDtypeStruct((2, NP, D), jnp.float32),
        jax.ShapeDtypeStruct((NP, H), jnp.float32),
        jax.ShapeDtypeStruct((NP, 1), jnp.float32),
    ],
)


BN2 = 1000


def _tc2_body(acc2_ref, inv_ref, z_ref, w2l_ref, c2_ref, out_ref):
    inv = inv_ref[...]
    m0 = acc2_ref[0] * inv
    m1 = acc2_ref[1] * inv
    lin = (jnp.dot(m0, w2l_ref[0], preferred_element_type=jnp.float32)
           + jnp.dot(m1, w2l_ref[1], preferred_element_type=jnp.float32)
           + z_ref[...] + c2_ref[...])
    out_ref[...] = jnp.maximum(lin, 0.0)


_tc2 = pl.pallas_call(
    _tc2_body,
    grid=(N // BN2,),
    in_specs=[
        pl.BlockSpec((2, BN2, D), lambda i: (0, i, 0)),
        pl.BlockSpec((BN2, 1), lambda i: (i, 0)),
        pl.BlockSpec((BN2, H), lambda i: (i, 0)),
        pl.BlockSpec((2, D, H), lambda i: (0, 0, 0)),
        pl.BlockSpec((1, H), lambda i: (0, 0)),
    ],
    out_specs=pl.BlockSpec((BN2, H), lambda i: (i, 0)),
    out_shape=jax.ShapeDtypeStruct((N, H), jnp.float32),
)


def kernel(x, edge_index, W1l, b1, W1r, bn1_g, bn1_b, W2l, b2, W2r,
           bn2_g, bn2_b):
    ei = edge_index.astype(jnp.int32)

    def chunked(n_pad, ch):
        pad = n_pad - E
        src_p = jnp.concatenate(
            [ei[0], jnp.arange(pad, dtype=jnp.int32) % N])
        dst_p = jnp.concatenate(
            [ei[1], N + (jnp.arange(pad, dtype=jnp.int32) % (NP - N))])
        return jnp.stack(
            [src_p.reshape(n_pad // ch, ch), dst_p.reshape(n_pad // ch, ch)],
            axis=1)

    ec = chunked(E2, CH)
    bn_inv = 1.0 / jnp.sqrt(1.0 + 1e-5)
    s1 = bn1_g * bn_inv
    s2 = bn2_g * bn_inv
    c1 = (b1 * s1 + bn1_b).reshape(1, H)
    c2 = (b2 * s2 + bn2_b).reshape(1, H)
    w1l = W1l.T * s1[None, :]
    w1r = W1r.T * s1[None, :]
    w2r = W2r.T * s2[None, :]
    w2l = (W2l.T * s2[None, :]).reshape(2, D, H)

    xp = jnp.concatenate([x, jnp.zeros((NP - N, D), jnp.float32)], axis=0)
    zeros_h = jnp.zeros((NP, D), jnp.float32)

    acc1, cnt = _agg1(xp, ec, zeros_h)
    h1s, z, inv = _tc1(acc1, cnt, xp, w1l, w1r, c1, w2r)
    acc2 = _agg2(h1s, ec, zeros_h)
    return _tc2(acc2, inv, z, w2l, c2)
